# baseline (speedup 1.0000x reference)
import functools

import jax
import jax.numpy as jnp
from jax import lax
from jax.experimental import pallas as pl
from jax.experimental.pallas import tpu as pltpu
from jax.experimental.pallas import tpu_sc as plsc

B = 128
I = 10000
O = 10000
H = 4
E = 160000

EB = 64
NBLK = E // EB
NSUB = 16
NCORE = 2
ROWS_PER_SUB = 624
TAIL_ROWS = O - NSUB * ROWS_PER_SUB


def _splat(vec, k):
    idx = jnp.full((16, 1), 0, jnp.int32) + k
    dn = lax.GatherDimensionNumbers(
        offset_dims=(), collapsed_slice_dims=(0,), start_index_map=(0,))
    return lax.gather(vec, idx, dn, slice_sizes=(1,),
                      mode=lax.GatherScatterMode.PROMISE_IN_BOUNDS)


def _sc_body(PHASE, xt, inids, outids, w1f, zeros_hbm, out_hbm,
             II, OI, WB, G, S, acc,
             is0, is1, is2, is3, gs0, gs1, ss0, ss1):
    cid = lax.axis_index("c")
    sid = lax.axis_index("s")
    r0 = sid * ROWS_PER_SUB
    IS = [is0, is1, is2, is3]
    GS = [gs0, gs1]
    SS = [ss0, ss1]

    nbps = NBLK // NSUB
    rem = NBLK - NSUB * nbps
    nblk = jnp.where(sid < rem, nbps + 1, nbps)
    base = sid * nbps + jnp.minimum(sid, rem)

    def issue_ids(i, f):
        e0 = (base + i) * EB
        pltpu.async_copy(inids.at[pl.ds(e0, EB)], II.at[f], IS[f])
        pltpu.async_copy(outids.at[pl.ds(e0, EB)], OI.at[f], IS[f])
        pltpu.async_copy(w1f.at[pl.ds(e0 * H, EB * H)], WB.at[f], IS[f])

    def wait_ids(i, f):
        e0 = (base + i) * EB
        pltpu.make_async_copy(inids.at[pl.ds(e0, EB)], II.at[f], IS[f]).wait()
        pltpu.make_async_copy(outids.at[pl.ds(e0, EB)], OI.at[f], IS[f]).wait()
        pltpu.make_async_copy(w1f.at[pl.ds(e0 * H, EB * H)], WB.at[f],
                              IS[f]).wait()

    jpass = 2 * cid + PHASE
    if True:
        if True:
            pltpu.sync_copy(zeros_hbm.at[pl.ds(r0, ROWS_PER_SUB)],
                            acc.at[pl.ds(r0, ROWS_PER_SUB)])

            @pl.when(sid == NSUB - 1)
            def _():
                pltpu.sync_copy(
                    zeros_hbm.at[pl.ds(NSUB * ROWS_PER_SUB, TAIL_ROWS)],
                    acc.at[pl.ds(NSUB * ROWS_PER_SUB, TAIL_ROWS)])
            plsc.subcore_barrier()

            issue_ids(0, 0)
            issue_ids(1, 1)
            wait_ids(0, 0)
            pltpu.async_copy(xt.at[II.at[0]], G.at[0], GS[0])

            @pl.loop(0, (NBLK // NSUB + 1 + 3) // 4)
            def _(k):
                for par in range(4):
                    i = 4 * k + par
                    p = par % 2
                    q = (par + 1) % 2
                    f = par
                    f1 = (par + 1) % 4
                    f2 = (par + 2) % 4

                    @pl.when(i < nblk)
                    def _(i=i, p=p, q=q, f=f, f1=f1, f2=f2):
                        @pl.when(i + 1 < nblk)
                        def _():
                            wait_ids(i + 1, f1)
                            pltpu.async_copy(xt.at[II.at[f1]], G.at[q], GS[q])

                        pltpu.make_async_copy(xt.at[II.at[f]], G.at[p],
                                              GS[p]).wait()

                        @pl.when(i >= 2)
                        def _():
                            pltpu.make_async_copy(S.at[p], acc.at[OI.at[f2]],
                                                  SS[p]).wait()

                        @pl.when(i + 2 < nblk)
                        def _():
                            issue_ids(i + 2, f2)

                        @pl.loop(0, EB // 4, unroll=4)
                        def _(g):
                            wv = WB[f, pl.ds(g * 16, 16)]
                            for t in range(4):
                                e = g * 4 + t
                                w = _splat(wv, t * H + jpass)
                                for h in range(8):
                                    S[p, e, pl.ds(h * 16, 16)] = (
                                        G[p, e, pl.ds(h * 16, 16)] * w)

                        pltpu.async_copy(S.at[p], acc.at[OI.at[f]], SS[p],
                                         add=True)

            pltpu.make_async_copy(S.at[0], acc.at[OI.at[0]], SS[0]).wait()
            pltpu.make_async_copy(S.at[1], acc.at[OI.at[1]], SS[1]).wait()

            plsc.subcore_barrier()
            pltpu.sync_copy(acc.at[pl.ds(r0, ROWS_PER_SUB)],
                            out_hbm.at[cid, pl.ds(r0, ROWS_PER_SUB)])

            @pl.when(sid == NSUB - 1)
            def _():
                pltpu.sync_copy(
                    acc.at[pl.ds(NSUB * ROWS_PER_SUB, TAIL_ROWS)],
                    out_hbm.at[cid, pl.ds(NSUB * ROWS_PER_SUB, TAIL_ROWS)])
            plsc.subcore_barrier()


def _sc_accumulate(phase, xt, in_ids, out_ids, w1f, zeros):
    mesh = plsc.VectorSubcoreMesh(core_axis_name="c", subcore_axis_name="s",
                                  num_cores=NCORE, num_subcores=NSUB)
    f = pl.kernel(
        functools.partial(_sc_body, phase),
        out_type=jax.ShapeDtypeStruct((NCORE, O, B), jnp.float32),
        mesh=mesh,
        name=f"sc_accum_phase{phase}",
        scratch_types=[
            pltpu.VMEM((4, EB), jnp.int32),
            pltpu.VMEM((4, EB), jnp.int32),
            pltpu.VMEM((4, EB * H), jnp.float32),
            pltpu.VMEM((2, EB, B), jnp.float32),
            pltpu.VMEM((2, EB, B), jnp.float32),
            pltpu.VMEM_SHARED((O, B), jnp.float32),
            pltpu.SemaphoreType.DMA,
            pltpu.SemaphoreType.DMA,
            pltpu.SemaphoreType.DMA,
            pltpu.SemaphoreType.DMA,
            pltpu.SemaphoreType.DMA,
            pltpu.SemaphoreType.DMA,
            pltpu.SemaphoreType.DMA,
            pltpu.SemaphoreType.DMA,
        ],
    )
    return f(xt, in_ids, out_ids, w1f, zeros)


OB = 1000


def _tc_post_a_body(hp_ref, w2_ref, th_ref, p_ref):
    h = jnp.tanh(hp_ref[...])
    th_ref[...] = h
    w2b = w2_ref[...]
    p_ref[...] = h[0] * w2b[:, 0:1] + h[1] * w2b[:, 1:2]


def _tc_post_b_body(hp_ref, w2_ref, pa_ref, th_ref, yr_ref):
    h = jnp.tanh(hp_ref[...])
    th_ref[...] = h
    w2b = w2_ref[...]
    p = pa_ref[...] + h[0] * w2b[:, 0:1] + h[1] * w2b[:, 1:2]
    yr_ref[...] = jnp.tanh(p)


def _tc_post_a(hp, w2p):
    return pl.pallas_call(
        _tc_post_a_body,
        grid=(O // OB,),
        in_specs=[
            pl.BlockSpec((NCORE, OB, B), lambda o: (0, o, 0)),
            pl.BlockSpec((OB, NCORE), lambda o: (o, 0)),
        ],
        out_specs=[
            pl.BlockSpec((NCORE, OB, B), lambda o: (0, o, 0)),
            pl.BlockSpec((OB, B), lambda o: (o, 0)),
        ],
        out_shape=[
            jax.ShapeDtypeStruct((NCORE, O, B), jnp.float32),
            jax.ShapeDtypeStruct((O, B), jnp.float32),
        ],
    )(hp, w2p)


def _tc_post_b(hp, w2p, pa):
    return pl.pallas_call(
        _tc_post_b_body,
        grid=(O // OB,),
        in_specs=[
            pl.BlockSpec((NCORE, OB, B), lambda o: (0, o, 0)),
            pl.BlockSpec((OB, NCORE), lambda o: (o, 0)),
            pl.BlockSpec((OB, B), lambda o: (o, 0)),
        ],
        out_specs=[
            pl.BlockSpec((NCORE, OB, B), lambda o: (0, o, 0)),
            pl.BlockSpec((OB, B), lambda o: (o, 0)),
        ],
        out_shape=[
            jax.ShapeDtypeStruct((NCORE, O, B), jnp.float32),
            jax.ShapeDtypeStruct((O, B), jnp.float32),
        ],
    )(hp, w2p, pa)


@jax.jit
def kernel(x, w1, w2, in_ids, out_ids):
    xt = x.T
    w1f = w1.reshape(-1)
    zeros = jnp.zeros((O, B), jnp.float32)

    hpa = _sc_accumulate(0, xt, in_ids, out_ids, w1f, zeros)
    hpb = _sc_accumulate(1, xt, in_ids, out_ids, w1f, zeros)
    tha, pa = _tc_post_a(hpa, w2[:, 0::2])
    thb, yr = _tc_post_b(hpb, w2[:, 1::2], pa)

    hidden = jnp.stack(
        [tha.transpose(2, 1, 0), thb.transpose(2, 1, 0)], axis=-1,
    ).reshape(B, O, H)
    y = yr.T
    return (y, hidden)


# speedup vs baseline: 1.1447x; 1.1447x over previous
import jax
import jax.numpy as jnp
from jax import lax
from jax.experimental import pallas as pl
from jax.experimental.pallas import tpu as pltpu
from jax.experimental.pallas import tpu_sc as plsc

B = 128
I = 10000
O = 10000
H = 4
E = 160000

EB = 64
NBLK = E // EB
NSUB = 16
NCORE = 2
ROWS_PER_SUB = 624
TAIL_ROWS = O - NSUB * ROWS_PER_SUB


def _splat(vec, k):
    idx = jnp.full((16, 1), k, jnp.int32)
    dn = lax.GatherDimensionNumbers(
        offset_dims=(), collapsed_slice_dims=(0,), start_index_map=(0,))
    return lax.gather(vec, idx, dn, slice_sizes=(1,),
                      mode=lax.GatherScatterMode.PROMISE_IN_BOUNDS)


def _sc_body(xt, inids, outids, w1f, zeros_hbm, out_hbm,
             II, OI, WB, G, S, acc,
             is0, is1, is2, is3, gs0, gs1, ss0, ss1):
    cid = lax.axis_index("c")
    sid = lax.axis_index("s")
    r0 = sid * ROWS_PER_SUB
    IS = [is0, is1, is2, is3]
    GS = [gs0, gs1]
    SS = [ss0, ss1]

    nbps = NBLK // NSUB
    rem = NBLK - NSUB * nbps
    nblk = jnp.where(sid < rem, nbps + 1, nbps)
    base = sid * nbps + jnp.minimum(sid, rem)

    def issue_ids(i, f, jpass):
        e0 = (base + i) * EB
        pltpu.async_copy(inids.at[pl.ds(e0, EB)], II.at[f], IS[f])
        pltpu.async_copy(outids.at[pl.ds(e0, EB)], OI.at[f], IS[f])
        pltpu.async_copy(w1f.at[pl.ds(jpass * E + e0, EB)], WB.at[f], IS[f])

    def wait_ids(i, f, jpass):
        e0 = (base + i) * EB
        pltpu.make_async_copy(inids.at[pl.ds(e0, EB)], II.at[f], IS[f]).wait()
        pltpu.make_async_copy(outids.at[pl.ds(e0, EB)], OI.at[f], IS[f]).wait()
        pltpu.make_async_copy(w1f.at[pl.ds(jpass * E + e0, EB)], WB.at[f],
                              IS[f]).wait()

    for jpass in range(H):
        @pl.when(cid == jpass // 2)
        def _(jpass=jpass):
            pltpu.sync_copy(zeros_hbm.at[pl.ds(r0, ROWS_PER_SUB)],
                            acc.at[pl.ds(r0, ROWS_PER_SUB)])

            @pl.when(sid == NSUB - 1)
            def _():
                pltpu.sync_copy(
                    zeros_hbm.at[pl.ds(NSUB * ROWS_PER_SUB, TAIL_ROWS)],
                    acc.at[pl.ds(NSUB * ROWS_PER_SUB, TAIL_ROWS)])
            plsc.subcore_barrier()

            issue_ids(0, 0, jpass)
            issue_ids(1, 1, jpass)
            wait_ids(0, 0, jpass)
            pltpu.async_copy(xt.at[II.at[0]], G.at[0], GS[0])

            @pl.loop(0, (NBLK // NSUB + 1 + 3) // 4)
            def _(k):
                for par in range(4):
                    i = 4 * k + par
                    p = par % 2
                    q = (par + 1) % 2
                    f = par
                    f1 = (par + 1) % 4
                    f2 = (par + 2) % 4

                    @pl.when(i < nblk)
                    def _(i=i, p=p, q=q, f=f, f1=f1, f2=f2):
                        @pl.when(i + 1 < nblk)
                        def _():
                            wait_ids(i + 1, f1, jpass)
                            pltpu.async_copy(xt.at[II.at[f1]], G.at[q], GS[q])

                        pltpu.make_async_copy(xt.at[II.at[f]], G.at[p],
                                              GS[p]).wait()

                        @pl.when(i >= 2)
                        def _():
                            pltpu.make_async_copy(S.at[p], acc.at[OI.at[f2]],
                                                  SS[p]).wait()

                        @pl.when(i + 2 < nblk)
                        def _():
                            issue_ids(i + 2, f2, jpass)

                        @pl.loop(0, EB // 16)
                        def _(g):
                            wv = WB[f, pl.ds(g * 16, 16)]
                            for t in range(16):
                                e = g * 16 + t
                                w = _splat(wv, t)
                                for h in range(8):
                                    S[p, e, pl.ds(h * 16, 16)] = (
                                        G[p, e, pl.ds(h * 16, 16)] * w)

                        pltpu.async_copy(S.at[p], acc.at[OI.at[f]], SS[p],
                                         add=True)

            pltpu.make_async_copy(S.at[0], acc.at[OI.at[0]], SS[0]).wait()
            pltpu.make_async_copy(S.at[1], acc.at[OI.at[1]], SS[1]).wait()

            plsc.subcore_barrier()
            pltpu.sync_copy(acc.at[pl.ds(r0, ROWS_PER_SUB)],
                            out_hbm.at[jpass, pl.ds(r0, ROWS_PER_SUB)])

            @pl.when(sid == NSUB - 1)
            def _():
                pltpu.sync_copy(
                    acc.at[pl.ds(NSUB * ROWS_PER_SUB, TAIL_ROWS)],
                    out_hbm.at[jpass, pl.ds(NSUB * ROWS_PER_SUB, TAIL_ROWS)])
            plsc.subcore_barrier()


def _sc_accumulate(xt, in_ids, out_ids, w1f, zeros):
    mesh = plsc.VectorSubcoreMesh(core_axis_name="c", subcore_axis_name="s",
                                  num_cores=NCORE, num_subcores=NSUB)
    f = pl.kernel(
        _sc_body,
        out_type=jax.ShapeDtypeStruct((H, O, B), jnp.float32),
        mesh=mesh,
        scratch_types=[
            pltpu.VMEM((4, EB), jnp.int32),
            pltpu.VMEM((4, EB), jnp.int32),
            pltpu.VMEM((4, EB), jnp.float32),
            pltpu.VMEM((2, EB, B), jnp.float32),
            pltpu.VMEM((2, EB, B), jnp.float32),
            pltpu.VMEM_SHARED((O, B), jnp.float32),
            pltpu.SemaphoreType.DMA,
            pltpu.SemaphoreType.DMA,
            pltpu.SemaphoreType.DMA,
            pltpu.SemaphoreType.DMA,
            pltpu.SemaphoreType.DMA,
            pltpu.SemaphoreType.DMA,
            pltpu.SemaphoreType.DMA,
            pltpu.SemaphoreType.DMA,
        ],
    )
    return f(xt, in_ids, out_ids, w1f, zeros)


OB = 1000


def _tc_body(hp_ref, w2_ref, th_ref, yr_ref):
    h = jnp.tanh(hp_ref[...])
    th_ref[...] = h
    w2b = w2_ref[...]
    acc = jnp.zeros((OB, B), jnp.float32)
    for j in range(H):
        acc = acc + h[j] * w2b[:, j:j + 1]
    yr_ref[...] = jnp.tanh(acc)


def _tc_post(hp, w2):
    return pl.pallas_call(
        _tc_body,
        grid=(O // OB,),
        in_specs=[
            pl.BlockSpec((H, OB, B), lambda o: (0, o, 0)),
            pl.BlockSpec((OB, H), lambda o: (o, 0)),
        ],
        out_specs=[
            pl.BlockSpec((H, OB, B), lambda o: (0, o, 0)),
            pl.BlockSpec((OB, B), lambda o: (o, 0)),
        ],
        out_shape=[
            jax.ShapeDtypeStruct((H, O, B), jnp.float32),
            jax.ShapeDtypeStruct((O, B), jnp.float32),
        ],
    )(hp, w2)


@jax.jit
def kernel(x, w1, w2, in_ids, out_ids):
    xt = x.T
    w1f = w1.T.reshape(-1)
    zeros = jnp.zeros((O, B), jnp.float32)

    hp = _sc_accumulate(xt, in_ids, out_ids, w1f, zeros)
    th, yr = _tc_post(hp, w2)

    hidden = th.transpose(2, 1, 0)
    y = yr.T
    return (y, hidden)
